# Initial kernel scaffold; baseline (speedup 1.0000x reference)
import functools

import jax
import jax.numpy as jnp
from jax import lax
from jax.experimental import pallas as pl
from jax.experimental.pallas import tpu as pltpu
from jax.experimental.pallas import tpu_sc as plsc

N = 10000
D = 128
E = 320000

NC = 2
NS = 16
NW = NC * NS
EPW = E // NW
CHUNK = 80
NCHUNK = EPW // CHUNK
RPT = N // NS

_MESH = dict(core_axis_name="c", subcore_axis_name="s")


@functools.partial(
    pl.kernel,
    mesh=plsc.VectorSubcoreMesh(**_MESH),
    out_type=jax.ShapeDtypeStruct((NC, N, 16), jnp.float32),
    scratch_types=[
        pltpu.VMEM((CHUNK,), jnp.int32),
        pltpu.VMEM((CHUNK, 16), jnp.float32),
        pltpu.VMEM_SHARED((N, 16), jnp.float32),
    ],
)
def _deg_pass(col_hbm, zeros_hbm, ones_hbm, out_hbm, idx_v, ones_v, acc_sh):
    c = lax.axis_index("c")
    s = lax.axis_index("s")
    wid = c * NS + s
    pltpu.sync_copy(ones_hbm, ones_v)
    pltpu.sync_copy(
        zeros_hbm.at[pl.ds(s * RPT, RPT)], acc_sh.at[pl.ds(s * RPT, RPT)]
    )
    plsc.subcore_barrier()

    def body(j, carry):
        off = wid * EPW + j * CHUNK
        pltpu.sync_copy(col_hbm.at[pl.ds(off, CHUNK)], idx_v)
        pltpu.sync_copy(ones_v, acc_sh.at[idx_v], add=True)
        return carry

    lax.fori_loop(0, NCHUNK, body, 0)
    plsc.subcore_barrier()
    pltpu.sync_copy(
        acc_sh.at[pl.ds(s * RPT, RPT)], out_hbm.at[c, pl.ds(s * RPT, RPT)]
    )


@functools.partial(
    pl.kernel,
    mesh=plsc.VectorSubcoreMesh(**_MESH),
    out_type=jax.ShapeDtypeStruct((NC, N, D), jnp.float32),
    scratch_types=[
        pltpu.VMEM((CHUNK,), jnp.int32),
        pltpu.VMEM((CHUNK,), jnp.int32),
        pltpu.VMEM((CHUNK,), jnp.int32),
        pltpu.VMEM((CHUNK,), jnp.int32),
        pltpu.VMEM((CHUNK, D), jnp.float32),
        pltpu.VMEM((CHUNK, D), jnp.float32),
        pltpu.VMEM_SHARED((N, D), jnp.float32),
        pltpu.SemaphoreType.DMA,
        pltpu.SemaphoreType.DMA,
    ],
)
def _edge_pass(h_hbm, row_hbm, col_hbm, zeros_hbm, out_hbm,
               src0_v, src1_v, dst0_v, dst1_v, rows0_v, rows1_v,
               acc_sh, sem0, sem1):
    c = lax.axis_index("c")
    s = lax.axis_index("s")
    wid = c * NS + s
    base = wid * EPW
    pltpu.sync_copy(
        zeros_hbm.at[pl.ds(s * RPT, RPT)], acc_sh.at[pl.ds(s * RPT, RPT)]
    )
    plsc.subcore_barrier()

    pltpu.sync_copy(row_hbm.at[pl.ds(base, CHUNK)], src0_v)
    pltpu.sync_copy(col_hbm.at[pl.ds(base, CHUNK)], dst0_v)
    pltpu.async_copy(h_hbm.at[src0_v], rows0_v, sem0)

    def body(j, carry):
        even = j % 2 == 0

        def step(src_v, dst_v, rows_v, sem, nsrc_v, ndst_v, nrows_v, nsem):
            @pl.when(j + 1 < NCHUNK)
            def _():
                noff = base + (j + 1) * CHUNK
                pltpu.sync_copy(row_hbm.at[pl.ds(noff, CHUNK)], nsrc_v)
                pltpu.sync_copy(col_hbm.at[pl.ds(noff, CHUNK)], ndst_v)
                pltpu.async_copy(h_hbm.at[nsrc_v], nrows_v, nsem)

            pltpu.make_async_copy(h_hbm.at[src_v], rows_v, sem).wait()
            pltpu.sync_copy(rows_v, acc_sh.at[dst_v], add=True)

        @pl.when(even)
        def _():
            step(src0_v, dst0_v, rows0_v, sem0, src1_v, dst1_v, rows1_v, sem1)

        @pl.when(jnp.logical_not(even))
        def _():
            step(src1_v, dst1_v, rows1_v, sem1, src0_v, dst0_v, rows0_v, sem0)

        return carry

    lax.fori_loop(0, NCHUNK, body, 0)
    plsc.subcore_barrier()
    pltpu.sync_copy(
        acc_sh.at[pl.ds(s * RPT, RPT)], out_hbm.at[c, pl.ds(s * RPT, RPT)]
    )


def _dinv_from_degp(degp):
    deg = 1.0 + degp[0, :, 0:1] + degp[1, :, 0:1]
    return lax.rsqrt(deg)


def _mm_body(degp_ref, x_ref, w_ref, o_ref):
    dinv = _dinv_from_degp(degp_ref[...])
    h = jnp.dot(x_ref[...], w_ref[...], preferred_element_type=jnp.float32)
    o_ref[...] = h * dinv


def _epi_mid_body(degp_ref, p_ref, h_ref, b_ref, g_ref, be_ref, a_ref,
                  w2_ref, o_ref):
    dinv = _dinv_from_degp(degp_ref[...])
    p = p_ref[...]
    pre = dinv * (p[0] + p[1] + h_ref[...]) + b_ref[...]
    mu = jnp.mean(pre, axis=0, keepdims=True)
    var = jnp.mean((pre - mu) ** 2, axis=0, keepdims=True)
    z = (pre - mu) * lax.rsqrt(var + 1e-5) * g_ref[...] + be_ref[...]
    z = jnp.where(z > 0, z, a_ref[0, 0] * z)
    o_ref[...] = dinv * jnp.dot(
        z, w2_ref[...], preferred_element_type=jnp.float32
    )


def _epi_fin_body(degp_ref, p_ref, h_ref, b_ref, g_ref, be_ref, a_ref, o_ref):
    dinv = _dinv_from_degp(degp_ref[...])
    p = p_ref[...]
    pre = dinv * (p[0] + p[1] + h_ref[...]) + b_ref[...]
    mu = jnp.mean(pre, axis=0, keepdims=True)
    var = jnp.mean((pre - mu) ** 2, axis=0, keepdims=True)
    z = (pre - mu) * lax.rsqrt(var + 1e-5) * g_ref[...] + be_ref[...]
    o_ref[...] = jnp.where(z > 0, z, a_ref[0, 0] * z)


_mm = pl.pallas_call(
    _mm_body, out_shape=jax.ShapeDtypeStruct((N, D), jnp.float32)
)
_epi_mid = pl.pallas_call(
    _epi_mid_body, out_shape=jax.ShapeDtypeStruct((N, D), jnp.float32)
)
_epi_fin = pl.pallas_call(
    _epi_fin_body, out_shape=jax.ShapeDtypeStruct((N, D), jnp.float32)
)


def kernel(x, edge_index, W1, b1, g1, be1, a1, W2, b2, g2, be2, a2):
    row = edge_index[0]
    col = edge_index[1]
    zeros_nd = jnp.zeros((N, D), jnp.float32)
    zeros_n16 = jnp.zeros((N, 16), jnp.float32)
    ones_c16 = jnp.ones((CHUNK, 16), jnp.float32)

    degp = _deg_pass(col, zeros_n16, ones_c16)
    h1 = _mm(degp, x, W1)
    p1 = _edge_pass(h1, row, col, zeros_nd)
    h2 = _epi_mid(degp, p1, h1, b1.reshape(1, D), g1.reshape(1, D),
                  be1.reshape(1, D), a1.reshape(1, 1), W2)
    p2 = _edge_pass(h2, row, col, zeros_nd)
    out = _epi_fin(degp, p2, h2, b2.reshape(1, D), g2.reshape(1, D),
                   be2.reshape(1, D), a2.reshape(1, 1))
    return out


# speedup vs baseline: 10.0994x; 10.0994x over previous
import functools

import jax
import jax.numpy as jnp
from jax import lax
from jax.experimental import pallas as pl
from jax.experimental.pallas import tpu as pltpu
from jax.experimental.pallas import tpu_sc as plsc

N = 10000
D = 128
E = 320000

NC = 2
NS = 16
NW = NC * NS
NP = 10240
RPT = NP // NS
CHUNK = 128
ERW = 80
EP = NW * ERW * CHUNK
_MESH = dict(core_axis_name="c", subcore_axis_name="s")


@functools.partial(
    pl.kernel,
    mesh=plsc.VectorSubcoreMesh(**_MESH),
    out_type=jax.ShapeDtypeStruct((NC, NP, D), jnp.float32),
    scratch_types=[
        pltpu.VMEM((ERW, CHUNK), jnp.int32),
        pltpu.VMEM((CHUNK, D), jnp.float32),
        pltpu.VMEM_SHARED((NP, D), jnp.float32),
    ],
)
def _deg_pass(col2_hbm, zeros_hbm, ones_hbm, out_hbm, dsts_v, ones_v, acc_sh):
    c = lax.axis_index("c")
    s = lax.axis_index("s")
    wid = c * NS + s
    pltpu.sync_copy(ones_hbm, ones_v)
    pltpu.sync_copy(col2_hbm.at[pl.ds(wid * ERW, ERW)], dsts_v)
    pltpu.sync_copy(
        zeros_hbm.at[pl.ds(s * RPT, RPT)], acc_sh.at[pl.ds(s * RPT, RPT)]
    )
    plsc.subcore_barrier()

    def body(j, carry):
        pltpu.sync_copy(ones_v, acc_sh.at[dsts_v.at[j]], add=True)
        return carry

    lax.fori_loop(0, ERW, body, 0)
    plsc.subcore_barrier()
    pltpu.sync_copy(
        acc_sh.at[pl.ds(s * RPT, RPT)], out_hbm.at[c, pl.ds(s * RPT, RPT)]
    )


@functools.partial(
    pl.kernel,
    mesh=plsc.VectorSubcoreMesh(**_MESH),
    out_type=jax.ShapeDtypeStruct((NC, NP, D), jnp.float32),
    scratch_types=[
        pltpu.VMEM((ERW // 2, CHUNK), jnp.int32),
        pltpu.VMEM((ERW // 2, CHUNK), jnp.int32),
        pltpu.VMEM((CHUNK, D), jnp.float32),
        pltpu.VMEM((CHUNK, D), jnp.float32),
        pltpu.VMEM_SHARED((NP, D), jnp.float32),
        pltpu.SemaphoreType.DMA,
        pltpu.SemaphoreType.DMA,
    ],
)
def _edge_pass(h_hbm, row2_hbm, col2_hbm, zeros_hbm, out_hbm,
               srcs_v, dsts_v, rows0_v, rows1_v, acc_sh, sem0, sem1):
    c = lax.axis_index("c")
    s = lax.axis_index("s")
    wid = c * NS + s
    HALF = ERW // 2
    pltpu.sync_copy(
        zeros_hbm.at[pl.ds(s * RPT, RPT)], acc_sh.at[pl.ds(s * RPT, RPT)]
    )
    plsc.subcore_barrier()

    for half in range(2):
        hbase = wid * ERW + half * HALF
        pltpu.sync_copy(row2_hbm.at[pl.ds(hbase, HALF)], srcs_v)
        pltpu.sync_copy(col2_hbm.at[pl.ds(hbase, HALF)], dsts_v)

        pltpu.async_copy(h_hbm.at[srcs_v.at[0]], rows0_v, sem0)

        def body(j, carry):
            even = j % 2 == 0

            def step(rows_v, sem, nrows_v, nsem):
                @pl.when(j + 1 < HALF)
                def _():
                    pltpu.async_copy(h_hbm.at[srcs_v.at[j + 1]], nrows_v, nsem)

                pltpu.make_async_copy(
                    h_hbm.at[srcs_v.at[j]], rows_v, sem
                ).wait()
                pltpu.sync_copy(rows_v, acc_sh.at[dsts_v.at[j]], add=True)

            @pl.when(even)
            def _():
                step(rows0_v, sem0, rows1_v, sem1)

            @pl.when(jnp.logical_not(even))
            def _():
                step(rows1_v, sem1, rows0_v, sem0)

            return carry

        lax.fori_loop(0, HALF, body, 0)

    plsc.subcore_barrier()
    pltpu.sync_copy(
        acc_sh.at[pl.ds(s * RPT, RPT)], out_hbm.at[c, pl.ds(s * RPT, RPT)]
    )


def _dinv_from_degs(degs):
    return lax.rsqrt(1.0 + degs)


def _mm_body(degp_ref, x_ref, w_ref, o_ref):
    dinv = _dinv_from_degs(degp_ref[...])
    h = jnp.dot(x_ref[...], w_ref[...], preferred_element_type=jnp.float32)
    o_ref[pl.ds(0, N), :] = h * dinv[:N]
    o_ref[pl.ds(N, NP - N), :] = jnp.zeros((NP - N, D), jnp.float32)


def _epi_mid_body(degp_ref, p_ref, h_ref, b_ref, g_ref, be_ref, a_ref,
                  w2_ref, o_ref):
    dinv = _dinv_from_degs(degp_ref[...])[:N]
    p = p_ref[...]
    h = h_ref[pl.ds(0, N), :]
    pre = dinv * (p[0, :N] + p[1, :N] + h) + b_ref[...]
    mu = jnp.mean(pre, axis=0, keepdims=True)
    var = jnp.mean((pre - mu) ** 2, axis=0, keepdims=True)
    z = (pre - mu) * lax.rsqrt(var + 1e-5) * g_ref[...] + be_ref[...]
    z = jnp.where(z > 0, z, a_ref[0, 0] * z)
    o_ref[pl.ds(0, N), :] = dinv * jnp.dot(
        z, w2_ref[...], preferred_element_type=jnp.float32
    )
    o_ref[pl.ds(N, NP - N), :] = jnp.zeros((NP - N, D), jnp.float32)


def _epi_fin_body(degp_ref, p_ref, h_ref, b_ref, g_ref, be_ref, a_ref, o_ref):
    dinv = _dinv_from_degs(degp_ref[...])[:N]
    p = p_ref[...]
    h = h_ref[pl.ds(0, N), :]
    pre = dinv * (p[0, :N] + p[1, :N] + h) + b_ref[...]
    mu = jnp.mean(pre, axis=0, keepdims=True)
    var = jnp.mean((pre - mu) ** 2, axis=0, keepdims=True)
    z = (pre - mu) * lax.rsqrt(var + 1e-5) * g_ref[...] + be_ref[...]
    o_ref[...] = jnp.where(z > 0, z, a_ref[0, 0] * z)


_mm = pl.pallas_call(
    _mm_body, out_shape=jax.ShapeDtypeStruct((NP, D), jnp.float32)
)
_epi_mid = pl.pallas_call(
    _epi_mid_body, out_shape=jax.ShapeDtypeStruct((NP, D), jnp.float32)
)
_epi_fin = pl.pallas_call(
    _epi_fin_body, out_shape=jax.ShapeDtypeStruct((N, D), jnp.float32)
)


def kernel(x, edge_index, W1, b1, g1, be1, a1, W2, b2, g2, be2, a2):
    pad = jnp.full((EP - E,), NP - 1, jnp.int32)
    row2 = jnp.concatenate([edge_index[0], pad]).reshape(ERW * NW, CHUNK)
    col2 = jnp.concatenate([edge_index[1], pad]).reshape(ERW * NW, CHUNK)
    zeros_nd = jnp.zeros((NP, D), jnp.float32)
    ones_cd = jnp.ones((CHUNK, D), jnp.float32)

    degp = _deg_pass(col2, zeros_nd, ones_cd)
    degs = (degp[0, :, 0] + degp[1, :, 0]).reshape(NP, 1)
    h1 = _mm(degs, x, W1)
    p1 = _edge_pass(h1, row2, col2, zeros_nd)
    h2 = _epi_mid(degs, p1, h1, b1.reshape(1, D), g1.reshape(1, D),
                  be1.reshape(1, D), a1.reshape(1, 1), W2)
    p2 = _edge_pass(h2, row2, col2, zeros_nd)
    out = _epi_fin(degs, p2, h2, b2.reshape(1, D), g2.reshape(1, D),
                   be2.reshape(1, D), a2.reshape(1, 1))
    return out


# speedup vs baseline: 26.8041x; 2.6540x over previous
import functools

import jax
import jax.numpy as jnp
from jax import lax
from jax.experimental import pallas as pl
from jax.experimental.pallas import tpu as pltpu
from jax.experimental.pallas import tpu_sc as plsc

N = 10000
D = 128
E = 320000

NC = 2
NS = 16
NW = NC * NS
NP = 10240
RPT = NP // NS
CHUNK = 128
ERW = 80
EP = NW * ERW * CHUNK
_MESH = dict(core_axis_name="c", subcore_axis_name="s")


@functools.partial(
    pl.kernel,
    mesh=plsc.VectorSubcoreMesh(**_MESH),
    out_type=jax.ShapeDtypeStruct((NC, NP, D), jnp.float32),
    scratch_types=[
        pltpu.VMEM((ERW, CHUNK), jnp.int32),
        pltpu.VMEM((CHUNK, D), jnp.float32),
        pltpu.VMEM_SHARED((NP, D), jnp.float32),
    ],
)
def _deg_pass(col2_hbm, zeros_hbm, ones_hbm, out_hbm, dsts_v, ones_v, acc_sh):
    c = lax.axis_index("c")
    s = lax.axis_index("s")
    wid = c * NS + s
    pltpu.sync_copy(ones_hbm, ones_v)
    pltpu.sync_copy(col2_hbm.at[pl.ds(wid * ERW, ERW)], dsts_v)
    pltpu.sync_copy(
        zeros_hbm.at[pl.ds(s * RPT, RPT)], acc_sh.at[pl.ds(s * RPT, RPT)]
    )
    plsc.subcore_barrier()

    def body(j, carry):
        pltpu.sync_copy(ones_v, acc_sh.at[dsts_v.at[j]], add=True)
        return carry

    lax.fori_loop(0, ERW, body, 0)
    plsc.subcore_barrier()
    pltpu.sync_copy(
        acc_sh.at[pl.ds(s * RPT, RPT)], out_hbm.at[c, pl.ds(s * RPT, RPT)]
    )


@functools.partial(
    pl.kernel,
    mesh=plsc.VectorSubcoreMesh(**_MESH),
    out_type=jax.ShapeDtypeStruct((NC, NP, D), jnp.float32),
    scratch_types=[
        pltpu.VMEM((ERW // 2, CHUNK), jnp.int32),
        pltpu.VMEM((ERW // 2, CHUNK), jnp.int32),
        pltpu.VMEM((CHUNK, D), jnp.float32),
        pltpu.VMEM((CHUNK, D), jnp.float32),
        pltpu.VMEM_SHARED((NP, D), jnp.float32),
        pltpu.SemaphoreType.DMA,
        pltpu.SemaphoreType.DMA,
    ],
)
def _edge_pass(h_hbm, row2_hbm, col2_hbm, zeros_hbm, out_hbm,
               srcs_v, dsts_v, rows0_v, rows1_v, acc_sh, sem0, sem1):
    c = lax.axis_index("c")
    s = lax.axis_index("s")
    wid = c * NS + s
    HALF = ERW // 2
    pltpu.sync_copy(
        zeros_hbm.at[pl.ds(s * RPT, RPT)], acc_sh.at[pl.ds(s * RPT, RPT)]
    )
    plsc.subcore_barrier()

    for half in range(2):
        hbase = wid * ERW + half * HALF
        pltpu.sync_copy(row2_hbm.at[pl.ds(hbase, HALF)], srcs_v)
        pltpu.sync_copy(col2_hbm.at[pl.ds(hbase, HALF)], dsts_v)

        pltpu.async_copy(h_hbm.at[srcs_v.at[0]], rows0_v, sem0)

        def body(j, carry):
            even = j % 2 == 0

            def step(rows_v, sem, nrows_v, nsem):
                @pl.when(j + 1 < HALF)
                def _():
                    pltpu.async_copy(h_hbm.at[srcs_v.at[j + 1]], nrows_v, nsem)

                pltpu.make_async_copy(
                    h_hbm.at[srcs_v.at[j]], rows_v, sem
                ).wait()
                pltpu.sync_copy(rows_v, acc_sh.at[dsts_v.at[j]], add=True)

            @pl.when(even)
            def _():
                step(rows0_v, sem0, rows1_v, sem1)

            @pl.when(jnp.logical_not(even))
            def _():
                step(rows1_v, sem1, rows0_v, sem0)

            return carry

        lax.fori_loop(0, HALF, body, 0)

    plsc.subcore_barrier()
    pltpu.sync_copy(
        acc_sh.at[pl.ds(s * RPT, RPT)], out_hbm.at[c, pl.ds(s * RPT, RPT)]
    )


def _dinv_from_degs(degs):
    return lax.rsqrt(1.0 + degs)


def _mm_body(degp_ref, x_ref, w_ref, o_ref):
    dinv = _dinv_from_degs(degp_ref[...])
    h = jnp.dot(x_ref[...], w_ref[...], preferred_element_type=jnp.float32)
    o_ref[pl.ds(0, N), :] = h * dinv[:N]
    o_ref[pl.ds(N, NP - N), :] = jnp.zeros((NP - N, D), jnp.float32)


def _epi_mid_body(degp_ref, p_ref, h_ref, b_ref, g_ref, be_ref, a_ref,
                  w2_ref, o_ref):
    dinv = _dinv_from_degs(degp_ref[...])[:N]
    p = p_ref[...]
    h = h_ref[pl.ds(0, N), :]
    pre = dinv * (p[0, :N] + p[1, :N] + h) + b_ref[...]
    mu = jnp.mean(pre, axis=0, keepdims=True)
    var = jnp.mean((pre - mu) ** 2, axis=0, keepdims=True)
    z = (pre - mu) * lax.rsqrt(var + 1e-5) * g_ref[...] + be_ref[...]
    z = jnp.where(z > 0, z, a_ref[0, 0] * z)
    o_ref[pl.ds(0, N), :] = dinv * jnp.dot(
        z, w2_ref[...], preferred_element_type=jnp.float32
    )
    o_ref[pl.ds(N, NP - N), :] = jnp.zeros((NP - N, D), jnp.float32)


def _epi_fin_body(degp_ref, p_ref, h_ref, b_ref, g_ref, be_ref, a_ref, o_ref):
    dinv = _dinv_from_degs(degp_ref[...])[:N]
    p = p_ref[...]
    h = h_ref[pl.ds(0, N), :]
    pre = dinv * (p[0, :N] + p[1, :N] + h) + b_ref[...]
    mu = jnp.mean(pre, axis=0, keepdims=True)
    var = jnp.mean((pre - mu) ** 2, axis=0, keepdims=True)
    z = (pre - mu) * lax.rsqrt(var + 1e-5) * g_ref[...] + be_ref[...]
    o_ref[...] = jnp.where(z > 0, z, a_ref[0, 0] * z)


_mm = pl.pallas_call(
    _mm_body, out_shape=jax.ShapeDtypeStruct((NP, D), jnp.float32)
)
_epi_mid = pl.pallas_call(
    _epi_mid_body, out_shape=jax.ShapeDtypeStruct((NP, D), jnp.float32)
)
_epi_fin = pl.pallas_call(
    _epi_fin_body, out_shape=jax.ShapeDtypeStruct((N, D), jnp.float32)
)


def kernel(x, edge_index, W1, b1, g1, be1, a1, W2, b2, g2, be2, a2):
    pad = (N + jnp.arange(EP - E, dtype=jnp.int32) % (NP - N)).astype(jnp.int32)
    row2 = jnp.concatenate([edge_index[0], pad]).reshape(ERW * NW, CHUNK)
    col2 = jnp.concatenate([edge_index[1], pad]).reshape(ERW * NW, CHUNK)
    zeros_nd = jnp.zeros((NP, D), jnp.float32)
    ones_cd = jnp.ones((CHUNK, D), jnp.float32)

    degp = _deg_pass(col2, zeros_nd, ones_cd)
    degs = (degp[0, :, 0] + degp[1, :, 0]).reshape(NP, 1)
    h1 = _mm(degs, x, W1)
    p1 = _edge_pass(h1, row2, col2, zeros_nd)
    h2 = _epi_mid(degs, p1, h1, b1.reshape(1, D), g1.reshape(1, D),
                  be1.reshape(1, D), a1.reshape(1, 1), W2)
    p2 = _edge_pass(h2, row2, col2, zeros_nd)
    out = _epi_fin(degs, p2, h2, b2.reshape(1, D), g2.reshape(1, D),
                   be2.reshape(1, D), a2.reshape(1, 1))
    return out
